# baseline (speedup 1.0000x reference)
import functools

import jax
import jax.numpy as jnp
from jax import lax
from jax.experimental import pallas as pl
from jax.experimental.pallas import tpu as pltpu
from jax.experimental.pallas import tpu_sc as plsc

_B = 16384
_D = 32


@functools.cache
def _build():
    info = plsc.get_sparse_core_info()
    nw = info.num_cores * info.num_subcores
    bpw = _B // nw
    nc = info.num_cores
    mesh = plsc.VectorSubcoreMesh(core_axis_name="c", subcore_axis_name="s")

    @functools.partial(
        pl.kernel,
        mesh=mesh,
        compiler_params=pltpu.CompilerParams(use_tc_tiling_on_sc=False),
        out_type=jax.ShapeDtypeStruct((2 * _B, _D), jnp.float32),
        scratch_types=[
            pltpu.VMEM((bpw,), jnp.int32),
            pltpu.VMEM((bpw,), jnp.int32),
            pltpu.VMEM((bpw, _D), jnp.float32),
            pltpu.VMEM((bpw, _D), jnp.float32),
            pltpu.VMEM((bpw,), jnp.int32),
            pltpu.VMEM((bpw,), jnp.int32),
            pltpu.SemaphoreType.DMA,
            pltpu.SemaphoreType.DMA,
        ],
    )
    def emb_kernel(sid_hbm, cid_hbm, semb_hbm, cemb_hbm, out_hbm,
                   sidx_v, cidx_v, srows_v, crows_v, oeidx_v, ooidx_v,
                   sem_s, sem_c):
        wid = lax.axis_index("s") * nc + lax.axis_index("c")
        base = wid * bpw
        pltpu.sync_copy(sid_hbm.at[pl.ds(base, bpw)], sidx_v)
        pltpu.sync_copy(cid_hbm.at[pl.ds(base, bpw)], cidx_v)
        cp_s = pltpu.async_copy(semb_hbm.at[sidx_v], srows_v, sem_s)
        cp_c = pltpu.async_copy(cemb_hbm.at[cidx_v], crows_v, sem_c)
        lanes = lax.iota(jnp.int32, 16)
        for k in range(bpw // 16):
            even = 2 * base + 2 * (k * 16) + 2 * lanes
            oeidx_v[pl.ds(k * 16, 16)] = even
            ooidx_v[pl.ds(k * 16, 16)] = even + 1
        cp_s.wait()
        cp_c.wait()
        pltpu.async_copy(srows_v, out_hbm.at[oeidx_v], sem_s).wait()
        pltpu.async_copy(crows_v, out_hbm.at[ooidx_v], sem_c).wait()

    return emb_kernel


def kernel(strategy_id, cause_index, strategy_emb, cause_emb):
    out2 = _build()(strategy_id.astype(jnp.int32),
                    cause_index.astype(jnp.int32),
                    strategy_emb, cause_emb)
    return out2.reshape(_B, 2 * _D)


# speedup vs baseline: 1.4162x; 1.4162x over previous
import functools

import jax
import jax.numpy as jnp
from jax import lax
from jax.experimental import pallas as pl
from jax.experimental.pallas import tpu as pltpu
from jax.experimental.pallas import tpu_sc as plsc

_B = 16384
_D = 32
_V = 100000
_VP = _V * _D // 128
_NCH = 4


@functools.cache
def _build():
    info = plsc.get_sparse_core_info()
    nw = info.num_cores * info.num_subcores
    bpw = _B // nw
    nc = info.num_cores
    chunk = bpw // _NCH
    mesh = plsc.VectorSubcoreMesh(core_axis_name="c", subcore_axis_name="s")

    @functools.partial(
        pl.kernel,
        mesh=mesh,
        compiler_params=pltpu.CompilerParams(use_tc_tiling_on_sc=True,
                                             needs_layout_passes=False),
        out_type=jax.ShapeDtypeStruct((_B, 2 * _D), jnp.float32),
        scratch_types=[
            pltpu.VMEM((bpw,), jnp.int32),
            pltpu.VMEM((bpw,), jnp.int32),
            pltpu.VMEM((bpw,), jnp.int32),
            pltpu.VMEM((8, _D), jnp.float32),
            pltpu.VMEM((chunk, 128), jnp.float32),
            pltpu.VMEM((chunk, 128), jnp.float32),
            pltpu.VMEM((chunk, 2 * _D), jnp.float32),
            pltpu.VMEM((chunk, 2 * _D), jnp.float32),
            pltpu.SemaphoreType.DMA,
            pltpu.SemaphoreType.DMA,
            pltpu.SemaphoreType.DMA,
            pltpu.SemaphoreType.DMA,
        ],
    )
    def lookup_kernel(sid_hbm, cid_hbm, semb_hbm, packed_hbm, out_hbm,
                      sidx_v, cidx_v, ci4_v, stab_v, crows_a, crows_b,
                      out_a, out_b, sem_a, sem_b, sem_oa, sem_ob):
        wid = lax.axis_index("s") * nc + lax.axis_index("c")
        base = wid * bpw
        pltpu.sync_copy(sid_hbm.at[pl.ds(base, bpw)], sidx_v)
        pltpu.sync_copy(cid_hbm.at[pl.ds(base, bpw)], cidx_v)
        pltpu.sync_copy(semb_hbm, stab_v)
        for g in range(bpw // 16):
            sl = pl.ds(g * 16, 16)
            ci4_v[sl] = lax.shift_right_logical(cidx_v[sl], 2)

        crows = (crows_a, crows_b)
        outs = (out_a, out_b)
        gsems = (sem_a, sem_b)
        osems = (sem_oa, sem_ob)
        lanes = lax.iota(jnp.int32, 16)

        def fire_gather(p):
            return pltpu.async_copy(
                packed_hbm.at[ci4_v.at[pl.ds(p * chunk, chunk)]],
                crows[p & 1].at[:], gsems[p & 1])

        def make_assemble(p):
            crow_v, out_v = crows[p & 1], outs[p & 1]

            def assemble(g, _):
                loc16 = g * 16 + lanes
                rows16 = p * chunk + loc16
                sid16 = plsc.load_gather(sidx_v, [rows16])
                cid16 = plsc.load_gather(cidx_v, [rows16])
                ccol = (cid16 & 3) * _D
                for d in range(_D):
                    dv = jnp.full((16,), d, jnp.int32)
                    sval = plsc.load_gather(stab_v, [sid16, dv])
                    plsc.store_scatter(out_v, [loc16, dv], sval)
                    cval = plsc.load_gather(crow_v, [loc16, ccol + d])
                    plsc.store_scatter(out_v, [loc16, dv + _D], cval)
                return _
            return assemble

        gcps = {0: fire_gather(0), 1: fire_gather(1)}
        ocps = {}
        for p in range(_NCH):
            gcps[p].wait()
            if p - 2 in ocps:
                ocps[p - 2].wait()
            lax.fori_loop(0, chunk // 16, make_assemble(p), 0)
            ocps[p] = pltpu.async_copy(
                outs[p & 1], out_hbm.at[pl.ds(base + p * chunk, chunk)],
                osems[p & 1])
            if p + 2 < _NCH:
                gcps[p + 2] = fire_gather(p + 2)
        ocps[_NCH - 2].wait()
        ocps[_NCH - 1].wait()

    return lookup_kernel


def kernel(strategy_id, cause_index, strategy_emb, cause_emb):
    packed = cause_emb.reshape(_VP, 128)
    return _build()(strategy_id.astype(jnp.int32),
                    cause_index.astype(jnp.int32),
                    strategy_emb, packed)
